# baseline (speedup 1.0000x reference)
import functools

import jax
import jax.numpy as jnp
from jax import lax
from jax.experimental import pallas as pl
from jax.experimental.pallas import tpu as pltpu

N = 10000
E = 320000
D = 128
H_BOND = 32

_EDGE_BLOCK = 4000


def _node_dense_body(x_ref, w1t_ref, b1_ref, w2t_ref, b2_ref,
                     w3at_ref, w3bt_ref, b3_ref,
                     val_ref, a_ref, b_ref):
    x = x_ref[...]
    h = jnp.maximum(
        jnp.dot(x, w1t_ref[...], preferred_element_type=jnp.float32)
        + b1_ref[...], 0.0)
    lv = jnp.dot(h, w2t_ref[...], preferred_element_type=jnp.float32) + b2_ref[...]
    m = jnp.max(lv, axis=-1, keepdims=True)
    e = jnp.exp(lv - m)
    val_ref[...] = e / jnp.sum(e, axis=-1, keepdims=True)
    a_ref[...] = jnp.dot(x, w3at_ref[...], preferred_element_type=jnp.float32) + b3_ref[...]
    b_ref[...] = jnp.dot(x, w3bt_ref[...], preferred_element_type=jnp.float32)


def _node_dense(x, W1t, b1, W2t, b2, W3at, W3bt, b3):
    return pl.pallas_call(
        _node_dense_body,
        out_shape=(
            jax.ShapeDtypeStruct((N, 8), jnp.float32),
            jax.ShapeDtypeStruct((N, H_BOND), jnp.float32),
            jax.ShapeDtypeStruct((N, H_BOND), jnp.float32),
        ),
    )(x, W1t, b1, W2t, b2, W3at, W3bt, b3)


def _edge_dense_body(g1_ref, g2_ref, w4t_ref, b4_ref, bond_ref):
    s = g1_ref[...] + g2_ref[...]
    hb = jnp.maximum(s, 0.0)
    l = jnp.dot(hb, w4t_ref[...], preferred_element_type=jnp.float32) + b4_ref[...]
    m = jnp.max(l, axis=-1, keepdims=True)
    e = jnp.exp(l - m)
    bond_ref[...] = e / jnp.sum(e, axis=-1, keepdims=True)


def _edge_dense(g1, g2, W4t, b4):
    nb = E // _EDGE_BLOCK
    return pl.pallas_call(
        _edge_dense_body,
        grid=(nb,),
        in_specs=[
            pl.BlockSpec((_EDGE_BLOCK, H_BOND), lambda i: (i, 0)),
            pl.BlockSpec((_EDGE_BLOCK, H_BOND), lambda i: (i, 0)),
            pl.BlockSpec((H_BOND, 4), lambda i: (0, 0)),
            pl.BlockSpec((1, 4), lambda i: (0, 0)),
        ],
        out_specs=pl.BlockSpec((_EDGE_BLOCK, 4), lambda i: (i, 0)),
        out_shape=jax.ShapeDtypeStruct((E, 4), jnp.float32),
    )(g1, g2, W4t, b4)


def _final_body(val_ref, deg2_ref, out_ref):
    v = val_ref[...]
    m = jnp.max(v, axis=1, keepdims=True)
    idx = lax.broadcasted_iota(jnp.int32, (N, 8), 1).astype(jnp.float32)
    pv = jnp.min(jnp.where(v == m, idx, 8.0), axis=1, keepdims=True) + 1.0
    deg = deg2_ref[:, 0:1] + deg2_ref[:, 1:2]
    diff = deg - pv
    out_ref[0, 0] = jnp.sum(diff * diff) * (1.0 / N)


def _final(valences, deg2t):
    return pl.pallas_call(
        _final_body,
        out_specs=pl.BlockSpec(memory_space=pltpu.SMEM),
        out_shape=jax.ShapeDtypeStruct((1, 1), jnp.float32),
    )(valences, deg2t)


def kernel(node_features, edge_index, W1, b1, W2, b2, W3, b3, W4, b4):
    x = node_features
    row = edge_index[0]
    col = edge_index[1]
    W3a = W3[:, :D]
    W3b = W3[:, D:]

    valences, A, B = _node_dense(
        x, W1.T, b1.reshape(1, 32), W2.T, b2.reshape(1, 8),
        W3a.T, W3b.T, b3.reshape(1, H_BOND))

    g1 = A[row]
    g2 = B[col]

    bond_types = _edge_dense(g1, g2, W4.T, b4.reshape(1, 4))

    w = (bond_types[:, 0] + 2.0 * bond_types[:, 1]
         + 3.0 * bond_types[:, 2] + 1.5 * bond_types[:, 3])
    deg = jnp.zeros((N,), jnp.float32).at[row].add(w)
    deg2 = jnp.stack([deg, jnp.zeros((N,), jnp.float32)], axis=1)

    violation = _final(valences, deg2)[0, 0]
    return (violation, valences, bond_types)


# speedup vs baseline: 1.6245x; 1.6245x over previous
import functools

import jax
import jax.numpy as jnp
from jax import lax
from jax.experimental import pallas as pl
from jax.experimental.pallas import tpu as pltpu
from jax.experimental.pallas import tpu_sc as plsc

N = 10000
E = 320000
D = 128
H_BOND = 32

_EDGE_BLOCK = 4000

_NC = 2
_NS = 16
_NW = _NC * _NS
_EPW = E // _NW
_GCH = 80
_GNCH = _EPW // _GCH


def _sc_gather_body(a_hbm, b_hbm, row_hbm, col_hbm, g1_hbm, g2_hbm,
                    row_v, col_v, bufa, bufb, sema, semb):
    wid = lax.axis_index("s") * _NC + lax.axis_index("c")
    base = wid * _EPW
    pltpu.sync_copy(row_hbm.at[pl.ds(base, _EPW)], row_v)
    pltpu.sync_copy(col_hbm.at[pl.ds(base, _EPW)], col_v)

    def body(i, carry):
        off = i * _GCH
        cpa = pltpu.async_copy(a_hbm.at[row_v.at[pl.ds(off, _GCH)]], bufa, sema)
        cpb = pltpu.async_copy(b_hbm.at[col_v.at[pl.ds(off, _GCH)]], bufb, semb)
        cpa.wait()
        cpb.wait()
        pltpu.sync_copy(bufa, g1_hbm.at[pl.ds(base + off, _GCH)])
        pltpu.sync_copy(bufb, g2_hbm.at[pl.ds(base + off, _GCH)])
        return carry

    lax.fori_loop(0, _GNCH, body, 0)


_sc_gather = functools.partial(
    pl.kernel,
    out_type=(jax.ShapeDtypeStruct((E, H_BOND), jnp.float32),
              jax.ShapeDtypeStruct((E, H_BOND), jnp.float32)),
    mesh=plsc.VectorSubcoreMesh(core_axis_name="c", subcore_axis_name="s"),
    scratch_types=[
        pltpu.VMEM((_EPW,), jnp.int32),
        pltpu.VMEM((_EPW,), jnp.int32),
        pltpu.VMEM((_GCH, H_BOND), jnp.float32),
        pltpu.VMEM((_GCH, H_BOND), jnp.float32),
        pltpu.SemaphoreType.DMA,
        pltpu.SemaphoreType.DMA,
    ],
    compiler_params=pltpu.CompilerParams(use_tc_tiling_on_sc=False),
)(_sc_gather_body)


def _node_dense_body(x_ref, w1t_ref, b1_ref, w2t_ref, b2_ref,
                     w3at_ref, w3bt_ref, b3_ref,
                     val_ref, a_ref, b_ref):
    x = x_ref[...]
    h = jnp.maximum(
        jnp.dot(x, w1t_ref[...], preferred_element_type=jnp.float32)
        + b1_ref[...], 0.0)
    lv = jnp.dot(h, w2t_ref[...], preferred_element_type=jnp.float32) + b2_ref[...]
    m = jnp.max(lv, axis=-1, keepdims=True)
    e = jnp.exp(lv - m)
    val_ref[...] = e / jnp.sum(e, axis=-1, keepdims=True)
    a_ref[...] = jnp.dot(x, w3at_ref[...], preferred_element_type=jnp.float32) + b3_ref[...]
    b_ref[...] = jnp.dot(x, w3bt_ref[...], preferred_element_type=jnp.float32)


def _node_dense(x, W1t, b1, W2t, b2, W3at, W3bt, b3):
    return pl.pallas_call(
        _node_dense_body,
        out_shape=(
            jax.ShapeDtypeStruct((N, 8), jnp.float32),
            jax.ShapeDtypeStruct((N, H_BOND), jnp.float32),
            jax.ShapeDtypeStruct((N, H_BOND), jnp.float32),
        ),
    )(x, W1t, b1, W2t, b2, W3at, W3bt, b3)


def _edge_dense_body(g1_ref, g2_ref, w4t_ref, b4_ref, bond_ref):
    s = g1_ref[...] + g2_ref[...]
    hb = jnp.maximum(s, 0.0)
    l = jnp.dot(hb, w4t_ref[...], preferred_element_type=jnp.float32) + b4_ref[...]
    m = jnp.max(l, axis=-1, keepdims=True)
    e = jnp.exp(l - m)
    bond_ref[...] = e / jnp.sum(e, axis=-1, keepdims=True)


def _edge_dense(g1, g2, W4t, b4):
    nb = E // _EDGE_BLOCK
    return pl.pallas_call(
        _edge_dense_body,
        grid=(nb,),
        in_specs=[
            pl.BlockSpec((_EDGE_BLOCK, H_BOND), lambda i: (i, 0)),
            pl.BlockSpec((_EDGE_BLOCK, H_BOND), lambda i: (i, 0)),
            pl.BlockSpec((H_BOND, 4), lambda i: (0, 0)),
            pl.BlockSpec((1, 4), lambda i: (0, 0)),
        ],
        out_specs=pl.BlockSpec((_EDGE_BLOCK, 4), lambda i: (i, 0)),
        out_shape=jax.ShapeDtypeStruct((E, 4), jnp.float32),
    )(g1, g2, W4t, b4)


def _final_body(val_ref, deg2_ref, out_ref):
    v = val_ref[...]
    m = jnp.max(v, axis=1, keepdims=True)
    idx = lax.broadcasted_iota(jnp.int32, (N, 8), 1).astype(jnp.float32)
    pv = jnp.min(jnp.where(v == m, idx, 8.0), axis=1, keepdims=True) + 1.0
    deg = deg2_ref[:, 0:1] + deg2_ref[:, 1:2]
    diff = deg - pv
    out_ref[0, 0] = jnp.sum(diff * diff) * (1.0 / N)


def _final(valences, deg2t):
    return pl.pallas_call(
        _final_body,
        out_specs=pl.BlockSpec(memory_space=pltpu.SMEM),
        out_shape=jax.ShapeDtypeStruct((1, 1), jnp.float32),
    )(valences, deg2t)


def kernel(node_features, edge_index, W1, b1, W2, b2, W3, b3, W4, b4):
    x = node_features
    row = edge_index[0]
    col = edge_index[1]
    W3a = W3[:, :D]
    W3b = W3[:, D:]

    valences, A, B = _node_dense(
        x, W1.T, b1.reshape(1, 32), W2.T, b2.reshape(1, 8),
        W3a.T, W3b.T, b3.reshape(1, H_BOND))

    g1, g2 = _sc_gather(A, B, row, col)

    bond_types = _edge_dense(g1, g2, W4.T, b4.reshape(1, 4))

    w = (bond_types[:, 0] + 2.0 * bond_types[:, 1]
         + 3.0 * bond_types[:, 2] + 1.5 * bond_types[:, 3])
    deg = jnp.zeros((N,), jnp.float32).at[row].add(w)
    deg2 = jnp.stack([deg, jnp.zeros((N,), jnp.float32)], axis=1)

    violation = _final(valences, deg2)[0, 0]
    return (violation, valences, bond_types)


# speedup vs baseline: 2.2762x; 1.4012x over previous
import functools

import jax
import jax.numpy as jnp
from jax import lax
from jax.experimental import pallas as pl
from jax.experimental.pallas import tpu as pltpu
from jax.experimental.pallas import tpu_sc as plsc

N = 10000
E = 320000
D = 128
H_BOND = 32

_EDGE_BLOCK = 4000

_NC = 2
_NS = 16
_NW = _NC * _NS
_EPW = E // _NW
_GCH = 80
_GNCH = _EPW // _GCH


def _sc_gather_body(a_hbm, b_hbm, row_hbm, col_hbm, g1_hbm, g2_hbm,
                    row_v, col_v, bufa, bufb, sema, semb):
    wid = lax.axis_index("s") * _NC + lax.axis_index("c")
    base = wid * _EPW
    pltpu.sync_copy(row_hbm.at[pl.ds(base, _EPW)], row_v)
    pltpu.sync_copy(col_hbm.at[pl.ds(base, _EPW)], col_v)

    def body(i, carry):
        off = i * _GCH
        cpa = pltpu.async_copy(a_hbm.at[row_v.at[pl.ds(off, _GCH)]], bufa, sema)
        cpb = pltpu.async_copy(b_hbm.at[col_v.at[pl.ds(off, _GCH)]], bufb, semb)
        cpa.wait()
        cpb.wait()
        pltpu.sync_copy(bufa, g1_hbm.at[pl.ds(base + off, _GCH)])
        pltpu.sync_copy(bufb, g2_hbm.at[pl.ds(base + off, _GCH)])
        return carry

    lax.fori_loop(0, _GNCH, body, 0)


_sc_gather = functools.partial(
    pl.kernel,
    out_type=(jax.ShapeDtypeStruct((E, H_BOND), jnp.float32),
              jax.ShapeDtypeStruct((E, H_BOND), jnp.float32)),
    mesh=plsc.VectorSubcoreMesh(core_axis_name="c", subcore_axis_name="s"),
    scratch_types=[
        pltpu.VMEM((_EPW,), jnp.int32),
        pltpu.VMEM((_EPW,), jnp.int32),
        pltpu.VMEM((_GCH, H_BOND), jnp.float32),
        pltpu.VMEM((_GCH, H_BOND), jnp.float32),
        pltpu.SemaphoreType.DMA,
        pltpu.SemaphoreType.DMA,
    ],
    compiler_params=pltpu.CompilerParams(use_tc_tiling_on_sc=False),
)(_sc_gather_body)


_SCH = 2000
_SNCH = _EPW // _SCH


def _sc_scatter_body(bond_hbm, row_hbm, zeros_hbm, part_hbm,
                     row_v, bond_v, w_v, deg_sh):
    c = lax.axis_index("c")
    s = lax.axis_index("s")
    wid = s * _NC + c
    base = wid * _EPW

    @pl.when(s == 0)
    def _():
        pltpu.sync_copy(zeros_hbm, deg_sh)

    plsc.subcore_barrier()

    def chunk(i, carry):
        off = i * _SCH
        pltpu.sync_copy(row_hbm.at[pl.ds(base + off, _SCH)], row_v)
        pltpu.sync_copy(bond_hbm.at[pl.ds(base + off, _SCH)], bond_v)

        def inner(k, carry2):
            e = k * 16 + lax.iota(jnp.int32, 16)
            b0 = plsc.load_gather(bond_v, [e, jnp.full((16,), 0, jnp.int32)])
            b1 = plsc.load_gather(bond_v, [e, jnp.full((16,), 1, jnp.int32)])
            b2 = plsc.load_gather(bond_v, [e, jnp.full((16,), 2, jnp.int32)])
            b3 = plsc.load_gather(bond_v, [e, jnp.full((16,), 3, jnp.int32)])
            w_v[pl.ds(k * 16, 16)] = b0 + 2.0 * b1 + 3.0 * b2 + 1.5 * b3
            return carry2

        lax.fori_loop(0, _SCH // 16, inner, 0)
        pltpu.sync_copy(w_v, deg_sh.at[row_v], add=True)
        return carry

    lax.fori_loop(0, _SNCH, chunk, 0)

    plsc.subcore_barrier()

    @pl.when(s == 0)
    def _():
        pltpu.sync_copy(deg_sh, part_hbm.at[c])


_sc_scatter = functools.partial(
    pl.kernel,
    out_type=jax.ShapeDtypeStruct((_NC, N), jnp.float32),
    mesh=plsc.VectorSubcoreMesh(core_axis_name="c", subcore_axis_name="s"),
    scratch_types=[
        pltpu.VMEM((_SCH,), jnp.int32),
        pltpu.VMEM((_SCH, 4), jnp.float32),
        pltpu.VMEM((_SCH,), jnp.float32),
        pltpu.VMEM_SHARED((N,), jnp.float32),
    ],
    compiler_params=pltpu.CompilerParams(use_tc_tiling_on_sc=False,
                                         needs_layout_passes=False),
)(_sc_scatter_body)


def _node_dense_body(x_ref, w1t_ref, b1_ref, w2t_ref, b2_ref,
                     w3at_ref, w3bt_ref, b3_ref,
                     val_ref, a_ref, b_ref):
    x = x_ref[...]
    h = jnp.maximum(
        jnp.dot(x, w1t_ref[...], preferred_element_type=jnp.float32)
        + b1_ref[...], 0.0)
    lv = jnp.dot(h, w2t_ref[...], preferred_element_type=jnp.float32) + b2_ref[...]
    m = jnp.max(lv, axis=-1, keepdims=True)
    e = jnp.exp(lv - m)
    val_ref[...] = e / jnp.sum(e, axis=-1, keepdims=True)
    a_ref[...] = jnp.dot(x, w3at_ref[...], preferred_element_type=jnp.float32) + b3_ref[...]
    b_ref[...] = jnp.dot(x, w3bt_ref[...], preferred_element_type=jnp.float32)


def _node_dense(x, W1t, b1, W2t, b2, W3at, W3bt, b3):
    return pl.pallas_call(
        _node_dense_body,
        out_shape=(
            jax.ShapeDtypeStruct((N, 8), jnp.float32),
            jax.ShapeDtypeStruct((N, H_BOND), jnp.float32),
            jax.ShapeDtypeStruct((N, H_BOND), jnp.float32),
        ),
    )(x, W1t, b1, W2t, b2, W3at, W3bt, b3)


def _edge_dense_body(g1_ref, g2_ref, w4t_ref, b4_ref, bond_ref):
    s = g1_ref[...] + g2_ref[...]
    hb = jnp.maximum(s, 0.0)
    l = jnp.dot(hb, w4t_ref[...], preferred_element_type=jnp.float32) + b4_ref[...]
    m = jnp.max(l, axis=-1, keepdims=True)
    e = jnp.exp(l - m)
    bond_ref[...] = e / jnp.sum(e, axis=-1, keepdims=True)


def _edge_dense(g1, g2, W4t, b4):
    nb = E // _EDGE_BLOCK
    return pl.pallas_call(
        _edge_dense_body,
        grid=(nb,),
        in_specs=[
            pl.BlockSpec((_EDGE_BLOCK, H_BOND), lambda i: (i, 0)),
            pl.BlockSpec((_EDGE_BLOCK, H_BOND), lambda i: (i, 0)),
            pl.BlockSpec((H_BOND, 4), lambda i: (0, 0)),
            pl.BlockSpec((1, 4), lambda i: (0, 0)),
        ],
        out_specs=pl.BlockSpec((_EDGE_BLOCK, 4), lambda i: (i, 0)),
        out_shape=jax.ShapeDtypeStruct((E, 4), jnp.float32),
    )(g1, g2, W4t, b4)


def _final_body(val_ref, deg2_ref, out_ref):
    v = val_ref[...]
    m = jnp.max(v, axis=1, keepdims=True)
    idx = lax.broadcasted_iota(jnp.int32, (N, 8), 1).astype(jnp.float32)
    pv = jnp.min(jnp.where(v == m, idx, 8.0), axis=1, keepdims=True) + 1.0
    deg = deg2_ref[:, 0:1] + deg2_ref[:, 1:2]
    diff = deg - pv
    out_ref[0, 0] = jnp.sum(diff * diff) * (1.0 / N)


def _final(valences, deg2t):
    return pl.pallas_call(
        _final_body,
        out_specs=pl.BlockSpec(memory_space=pltpu.SMEM),
        out_shape=jax.ShapeDtypeStruct((1, 1), jnp.float32),
    )(valences, deg2t)


def kernel(node_features, edge_index, W1, b1, W2, b2, W3, b3, W4, b4):
    x = node_features
    row = edge_index[0]
    col = edge_index[1]
    W3a = W3[:, :D]
    W3b = W3[:, D:]

    valences, A, B = _node_dense(
        x, W1.T, b1.reshape(1, 32), W2.T, b2.reshape(1, 8),
        W3a.T, W3b.T, b3.reshape(1, H_BOND))

    g1, g2 = _sc_gather(A, B, row, col)

    bond_types = _edge_dense(g1, g2, W4.T, b4.reshape(1, 4))

    part = _sc_scatter(bond_types, row, jnp.zeros((N,), jnp.float32))
    deg2 = part.T

    violation = _final(valences, deg2)[0, 0]
    return (violation, valences, bond_types)


# speedup vs baseline: 2.9324x; 1.2883x over previous
import functools

import jax
import jax.numpy as jnp
from jax import lax
from jax.experimental import pallas as pl
from jax.experimental.pallas import tpu as pltpu
from jax.experimental.pallas import tpu_sc as plsc

N = 10000
E = 320000
D = 128
H_BOND = 32

_NC = 2
_NS = 16
_NW = _NC * _NS
_EPW = E // _NW
_CH = 80
_NCH = _EPW // _CH
_NG = _CH // 16


def _sc_edge_body(a_hbm, b_hbm, row_hbm, col_hbm, w4_hbm, b4_hbm, zeros_hbm,
                  bond_hbm, part_hbm,
                  row_v, col_v, bufa, bufb, w4_v, b4_v, bond_v, w_v, deg_sh,
                  sema, semb):
    c = lax.axis_index("c")
    s = lax.axis_index("s")
    wid = s * _NC + c
    base = wid * _EPW

    @pl.when(s == 0)
    def _():
        pltpu.sync_copy(zeros_hbm, deg_sh)

    pltpu.sync_copy(w4_hbm, w4_v)
    pltpu.sync_copy(b4_hbm, b4_v)
    plsc.subcore_barrier()

    def chunk(i, carry):
        off = i * _CH
        pltpu.sync_copy(row_hbm.at[pl.ds(base + off, _CH)], row_v)
        pltpu.sync_copy(col_hbm.at[pl.ds(base + off, _CH)], col_v)
        cpa = pltpu.async_copy(a_hbm.at[row_v], bufa, sema)
        cpb = pltpu.async_copy(b_hbm.at[col_v], bufb, semb)
        cpa.wait()
        cpb.wait()

        lanes = lax.iota(jnp.int32, 16)
        b4s = [b4_v[j] for j in range(4)]
        acc0 = [b4s[j] for j in range(4)]
        accs = []
        for g in range(_NG):
            accs.extend(acc0)

        def kstep(kk, accs):
            accs = list(accs)
            w4s = [w4_v[j, kk] for j in range(4)]
            kkv = jnp.full((16,), 0, jnp.int32) + kk
            for g in range(_NG):
                e = g * 16 + lanes
                av = plsc.load_gather(bufa, [e, kkv])
                bv = plsc.load_gather(bufb, [e, kkv])
                sv = jnp.maximum(av + bv, 0.0)
                for j in range(4):
                    accs[g * 4 + j] = accs[g * 4 + j] + w4s[j] * sv
            return tuple(accs)

        accs = lax.fori_loop(0, H_BOND, kstep, tuple(accs))

        for g in range(_NG):
            l0, l1, l2, l3 = accs[g * 4:g * 4 + 4]
            m = jnp.maximum(jnp.maximum(l0, l1), jnp.maximum(l2, l3))
            e0 = jnp.exp(l0 - m)
            e1 = jnp.exp(l1 - m)
            e2 = jnp.exp(l2 - m)
            e3 = jnp.exp(l3 - m)
            inv = 1.0 / (e0 + e1 + e2 + e3)
            bond_v[0, pl.ds(g * 16, 16)] = e0 * inv
            bond_v[1, pl.ds(g * 16, 16)] = e1 * inv
            bond_v[2, pl.ds(g * 16, 16)] = e2 * inv
            bond_v[3, pl.ds(g * 16, 16)] = e3 * inv
            w_v[pl.ds(g * 16, 16)] = (e0 + 2.0 * e1 + 3.0 * e2 + 1.5 * e3) * inv

        for j in range(4):
            pltpu.sync_copy(bond_v.at[j], bond_hbm.at[j, pl.ds(base + off, _CH)])
        pltpu.sync_copy(w_v, deg_sh.at[row_v], add=True)
        return carry

    lax.fori_loop(0, _NCH, chunk, 0)

    plsc.subcore_barrier()

    @pl.when(s == 0)
    def _():
        pltpu.sync_copy(deg_sh, part_hbm.at[c])


_sc_edge = functools.partial(
    pl.kernel,
    out_type=(jax.ShapeDtypeStruct((4, E), jnp.float32),
              jax.ShapeDtypeStruct((_NC, N), jnp.float32)),
    mesh=plsc.VectorSubcoreMesh(core_axis_name="c", subcore_axis_name="s"),
    scratch_types=[
        pltpu.VMEM((_CH,), jnp.int32),
        pltpu.VMEM((_CH,), jnp.int32),
        pltpu.VMEM((_CH, H_BOND), jnp.float32),
        pltpu.VMEM((_CH, H_BOND), jnp.float32),
        pltpu.VMEM((4, H_BOND, 16), jnp.float32),
        pltpu.VMEM((4, 16), jnp.float32),
        pltpu.VMEM((4, _CH), jnp.float32),
        pltpu.VMEM((_CH,), jnp.float32),
        pltpu.VMEM_SHARED((N,), jnp.float32),
        pltpu.SemaphoreType.DMA,
        pltpu.SemaphoreType.DMA,
    ],
    compiler_params=pltpu.CompilerParams(use_tc_tiling_on_sc=False,
                                         needs_layout_passes=False),
)(_sc_edge_body)


def _node_dense_body(x_ref, w1t_ref, b1_ref, w2t_ref, b2_ref,
                     w3at_ref, w3bt_ref, b3_ref,
                     val_ref, a_ref, b_ref):
    x = x_ref[...]
    h = jnp.maximum(
        jnp.dot(x, w1t_ref[...], preferred_element_type=jnp.float32)
        + b1_ref[...], 0.0)
    lv = jnp.dot(h, w2t_ref[...], preferred_element_type=jnp.float32) + b2_ref[...]
    m = jnp.max(lv, axis=-1, keepdims=True)
    e = jnp.exp(lv - m)
    val_ref[...] = e / jnp.sum(e, axis=-1, keepdims=True)
    a_ref[...] = jnp.dot(x, w3at_ref[...], preferred_element_type=jnp.float32) + b3_ref[...]
    b_ref[...] = jnp.dot(x, w3bt_ref[...], preferred_element_type=jnp.float32)


def _node_dense(x, W1t, b1, W2t, b2, W3at, W3bt, b3):
    return pl.pallas_call(
        _node_dense_body,
        out_shape=(
            jax.ShapeDtypeStruct((N, 8), jnp.float32),
            jax.ShapeDtypeStruct((N, H_BOND), jnp.float32),
            jax.ShapeDtypeStruct((N, H_BOND), jnp.float32),
        ),
    )(x, W1t, b1, W2t, b2, W3at, W3bt, b3)


def _final_body(val_ref, deg2_ref, out_ref):
    v = val_ref[...]
    m = jnp.max(v, axis=1, keepdims=True)
    idx = lax.broadcasted_iota(jnp.int32, (N, 8), 1).astype(jnp.float32)
    pv = jnp.min(jnp.where(v == m, idx, 8.0), axis=1, keepdims=True) + 1.0
    deg = deg2_ref[:, 0:1] + deg2_ref[:, 1:2]
    diff = deg - pv
    out_ref[0, 0] = jnp.sum(diff * diff) * (1.0 / N)


def _final(valences, deg2t):
    return pl.pallas_call(
        _final_body,
        out_specs=pl.BlockSpec(memory_space=pltpu.SMEM),
        out_shape=jax.ShapeDtypeStruct((1, 1), jnp.float32),
    )(valences, deg2t)


def kernel(node_features, edge_index, W1, b1, W2, b2, W3, b3, W4, b4):
    x = node_features
    row = edge_index[0]
    col = edge_index[1]
    W3a = W3[:, :D]
    W3b = W3[:, D:]

    valences, A, B = _node_dense(
        x, W1.T, b1.reshape(1, 32), W2.T, b2.reshape(1, 8),
        W3a.T, W3b.T, b3.reshape(1, H_BOND))

    w4bc = jnp.broadcast_to(W4[:, :, None], (4, H_BOND, 16))
    b4bc = jnp.broadcast_to(b4[:, None], (4, 16))

    bond_t, part = _sc_edge(A, B, row, col, w4bc, b4bc,
                            jnp.zeros((N,), jnp.float32))

    bond_types = bond_t.T
    deg2 = part.T

    violation = _final(valences, deg2)[0, 0]
    return (violation, valences, bond_types)
